# baseline (speedup 1.0000x reference)
import dataclasses
import functools

import jax
import jax.numpy as jnp
from jax import lax
from jax.experimental import pallas as pl
from jax.experimental.pallas import tpu as pltpu
from jax.experimental.pallas import tpu_sc as plsc

N = 100000
D = 128
G = 512
NC = 2
NS = 16
NW = NC * NS
RPW = N // NW
STRIPE = G // NS
L = 16
NVEC = (RPW + L - 1) // L
IDS_PAD = NVEC * L

BLK = 125
N_SC = 36000
NBLK = N_SC // (NW * BLK)
TCB = 2000
N_TC = N - N_SC
NTCB = N_TC // TCB


def _seg_pool_sc(node_r, batch_r, batch_f, zeros_d):
    mesh = plsc.VectorSubcoreMesh(core_axis_name="c", subcore_axis_name="s")
    cp = pltpu.CompilerParams()
    if "needs_layout_passes" in pltpu.CompilerParams.__dataclass_fields__:
        cp = dataclasses.replace(cp, needs_layout_passes=False)

    @functools.partial(
        pl.kernel,
        compiler_params=cp,
        out_type=(
            jax.ShapeDtypeStruct((NC, G, D), jnp.float32),
            jax.ShapeDtypeStruct((NW, G), jnp.float32),
        ),
        mesh=mesh,
        scratch_types=[
            pltpu.VMEM((NBLK, BLK), jnp.int32),
            pltpu.VMEM((IDS_PAD,), jnp.int32),
            pltpu.VMEM((BLK, D), jnp.float32),
            pltpu.VMEM((BLK, D), jnp.float32),
            pltpu.VMEM((STRIPE, D), jnp.float32),
            pltpu.VMEM((G,), jnp.float32),
            pltpu.VMEM((L,), jnp.int32),
            pltpu.VMEM_SHARED((G, D), jnp.float32),
            pltpu.SemaphoreType.DMA,
            pltpu.SemaphoreType.DMA,
            pltpu.SemaphoreType.DMA,
            pltpu.SemaphoreType.DMA,
        ],
    )
    def k(node_hbm, batch_hbm, batchf_hbm, zd_hbm, sum_hbm, cnt_hbm,
          ids_v, idsf_v, buf0_v, buf1_v, zed_v, hist_v, cm_v, acc_sh,
          sin0, sin1, ssc0, ssc1):
        c = lax.axis_index("c")
        s = lax.axis_index("s")
        wid = c * NS + s
        bufs = (buf0_v, buf1_v)
        sins = (sin0, sin1)
        sscs = (ssc0, ssc1)

        in_dma = {}
        sc_dma = {}

        def start_in(j):
            in_dma[j] = pltpu.async_copy(node_hbm.at[wid, j], bufs[j % 2],
                                         sins[j % 2])

        def start_sc(j):
            sc_dma[j] = pltpu.async_copy(bufs[j % 2], acc_sh.at[ids_v.at[j]],
                                         sscs[j % 2], add=True)

        start_in(0)
        start_in(1)

        pltpu.sync_copy(zd_hbm, zed_v)
        pltpu.sync_copy(zed_v, acc_sh.at[pl.ds(s * STRIPE, STRIPE)])
        zf = jnp.zeros((L,), jnp.float32)

        @pl.loop(0, G // L)
        def _(i):
            hist_v[pl.ds(i * L, L)] = zf

        pltpu.sync_copy(batch_hbm.at[wid], ids_v)
        pltpu.sync_copy(batchf_hbm.at[wid], idsf_v)

        lanes = lax.iota(jnp.int32, L)

        def rle_step(i, prev):
            base = pl.multiple_of(i * L, L)
            a = idsf_v[pl.ds(base, L)]
            gpos = base + lanes
            nxt = plsc.load_gather(idsf_v, [jnp.minimum(gpos + 1, RPW - 1)])
            valid = gpos < RPW
            is_end = ((a != nxt) | (gpos == RPW - 1)) & valid
            e = jnp.where(is_end, gpos, -1)
            cm = plsc.cummax(e)
            cm_v[...] = cm
            shifted = plsc.load_gather(cm_v, [jnp.maximum(lanes - 1, 0)])
            prev_v = jnp.where(lanes == 0, prev, shifted)
            prev_v = jnp.maximum(prev_v, prev)
            runlen = (gpos - prev_v).astype(jnp.float32)
            plsc.addupdate_scatter(hist_v, [a], runlen, mask=is_end)
            return jnp.maximum(prev, jnp.max(e))

        lax.fori_loop(0, NVEC, rle_step, jnp.int32(-1))
        pltpu.sync_copy(hist_v, cnt_hbm.at[wid])

        plsc.subcore_barrier()

        for j in range(NBLK):
            in_dma[j].wait()
            start_sc(j)
            sc_dma[j].wait()
            if j + 2 < NBLK:
                start_in(j + 2)

        plsc.subcore_barrier()

        pltpu.sync_copy(acc_sh.at[pl.ds(s * STRIPE, STRIPE)], zed_v)
        pltpu.sync_copy(zed_v, sum_hbm.at[c, pl.ds(s * STRIPE, STRIPE)])

    return k(node_r, batch_r, batch_f, zeros_d)


def _onehot_partial_tc(node_tc, ids_tc):
    def body(ids_ref, node_ref, out_ref, acc_ref):
        i = pl.program_id(0)

        @pl.when(i == 0)
        def _():
            acc_ref[...] = jnp.zeros_like(acc_ref)

        ids = ids_ref[0]
        gi = lax.broadcasted_iota(jnp.int32, (G, TCB), 0)
        oh = jnp.where(ids == gi, 1.0, 0.0)
        acc_ref[...] += lax.dot_general(
            oh, node_ref[...], (((1,), (0,)), ((), ())),
            preferred_element_type=jnp.float32)

        @pl.when(i == NTCB - 1)
        def _():
            out_ref[...] = acc_ref[...]

    return pl.pallas_call(
        body,
        grid=(NTCB,),
        in_specs=[
            pl.BlockSpec((1, 1, TCB), lambda i: (i, 0, 0)),
            pl.BlockSpec((TCB, D), lambda i: (i, 0)),
        ],
        out_specs=pl.BlockSpec((G, D), lambda i: (0, 0)),
        out_shape=jax.ShapeDtypeStruct((G, D), jnp.float32),
        scratch_shapes=[pltpu.VMEM((G, D), jnp.float32)],
    )(ids_tc, node_tc)


def _mlp_tc(parts, tc_part, cnt, W1, b1, W2, b2, Wout, bout):
    def body(part_ref, tcp_ref, cnt_ref, ones_ref, w1_ref, b1_ref, w2_ref,
             b2_ref, wo_ref, bo_ref, out_ref):
        seg = part_ref[0] + part_ref[1] + tcp_ref[...]
        cnt_col = lax.dot_general(
            cnt_ref[...], ones_ref[...], (((0,), (0,)), ((), ())),
            preferred_element_type=jnp.float32,
            precision=lax.Precision.HIGHEST)
        h = seg / jnp.maximum(cnt_col, 1.0)
        h = jax.nn.softplus(
            lax.dot_general(h, w1_ref[...], (((1,), (1,)), ((), ())),
                            preferred_element_type=jnp.float32) + b1_ref[...])
        h = jax.nn.softplus(
            lax.dot_general(h, w2_ref[...], (((1,), (1,)), ((), ())),
                            preferred_element_type=jnp.float32) + b2_ref[...])
        hh = h.astype(jnp.bfloat16).astype(jnp.float32)
        ww = wo_ref[...].astype(jnp.bfloat16).astype(jnp.float32)
        out = jnp.sum(hh * ww, axis=1, keepdims=True) + bo_ref[0, 0]
        out_ref[...] = out

    ones_col = jnp.ones((NW, 1), jnp.float32)
    return pl.pallas_call(
        body,
        out_shape=jax.ShapeDtypeStruct((G, 1), jnp.float32),
    )(parts, tc_part, cnt, ones_col, W1, b1.reshape(1, D),
      W2, b2.reshape(1, D), Wout, bout.reshape(1, 1))


def kernel(node_attr, batch, W1, b1, W2, b2, Wout, bout):
    batch_i = batch.astype(jnp.int32)
    node_r = node_attr[:N_SC].reshape(NW, NBLK, BLK, D)
    batch_r = batch_i[:N_SC].reshape(NW, NBLK, BLK)
    batch_f = jnp.pad(batch_i.reshape(NW, RPW), ((0, 0), (0, IDS_PAD - RPW)))
    zeros_d = jnp.zeros((STRIPE, D), jnp.float32)
    node_tc = node_attr[N_SC:]
    ids_tc = batch_i[N_SC:].reshape(NTCB, 1, TCB)
    parts, cnt = _seg_pool_sc(node_r, batch_r, batch_f, zeros_d)
    tc_part = _onehot_partial_tc(node_tc, ids_tc)
    out = _mlp_tc(parts, tc_part, cnt, W1, b1, W2, b2, Wout, bout)
    return out.reshape(-1)


# speedup vs baseline: 1.0002x; 1.0002x over previous
import dataclasses
import functools

import jax
import jax.numpy as jnp
from jax import lax
from jax.experimental import pallas as pl
from jax.experimental.pallas import tpu as pltpu
from jax.experimental.pallas import tpu_sc as plsc

N = 100000
D = 128
G = 512
NC = 2
NS = 16
NW = NC * NS
RPW = N // NW
STRIPE = G // NS
L = 16
NVEC = (RPW + L - 1) // L
IDS_PAD = NVEC * L

BLK = 125
N_SC = 36000
NBLK = N_SC // (NW * BLK)
TCB = 2000
N_TC = N - N_SC
NTCB = N_TC // TCB


def _seg_pool_sc(node_r, batch_r, batch_f, zeros_d):
    mesh = plsc.VectorSubcoreMesh(core_axis_name="c", subcore_axis_name="s")
    cp = pltpu.CompilerParams()
    if "needs_layout_passes" in pltpu.CompilerParams.__dataclass_fields__:
        cp = dataclasses.replace(cp, needs_layout_passes=False)
    if "has_side_effects" in pltpu.CompilerParams.__dataclass_fields__:
        cp = dataclasses.replace(cp, has_side_effects=False)

    @functools.partial(
        pl.kernel,
        compiler_params=cp,
        out_type=(
            jax.ShapeDtypeStruct((NC, G, D), jnp.float32),
            jax.ShapeDtypeStruct((NW, G), jnp.float32),
        ),
        mesh=mesh,
        scratch_types=[
            pltpu.VMEM((NBLK, BLK), jnp.int32),
            pltpu.VMEM((IDS_PAD,), jnp.int32),
            pltpu.VMEM((BLK, D), jnp.float32),
            pltpu.VMEM((BLK, D), jnp.float32),
            pltpu.VMEM((STRIPE, D), jnp.float32),
            pltpu.VMEM((G,), jnp.float32),
            pltpu.VMEM((L,), jnp.int32),
            pltpu.VMEM_SHARED((G, D), jnp.float32),
            pltpu.SemaphoreType.DMA,
            pltpu.SemaphoreType.DMA,
            pltpu.SemaphoreType.DMA,
            pltpu.SemaphoreType.DMA,
        ],
    )
    def k(node_hbm, batch_hbm, batchf_hbm, zd_hbm, sum_hbm, cnt_hbm,
          ids_v, idsf_v, buf0_v, buf1_v, zed_v, hist_v, cm_v, acc_sh,
          sin0, sin1, ssc0, ssc1):
        c = lax.axis_index("c")
        s = lax.axis_index("s")
        wid = c * NS + s
        bufs = (buf0_v, buf1_v)
        sins = (sin0, sin1)
        sscs = (ssc0, ssc1)

        in_dma = {}
        sc_dma = {}

        def start_in(j):
            in_dma[j] = pltpu.async_copy(node_hbm.at[wid, j], bufs[j % 2],
                                         sins[j % 2])

        def start_sc(j):
            sc_dma[j] = pltpu.async_copy(bufs[j % 2], acc_sh.at[ids_v.at[j]],
                                         sscs[j % 2], add=True)

        start_in(0)
        start_in(1)

        pltpu.sync_copy(zd_hbm, zed_v)
        pltpu.sync_copy(zed_v, acc_sh.at[pl.ds(s * STRIPE, STRIPE)])
        zf = jnp.zeros((L,), jnp.float32)

        @pl.loop(0, G // L)
        def _(i):
            hist_v[pl.ds(i * L, L)] = zf

        pltpu.sync_copy(batch_hbm.at[wid], ids_v)
        pltpu.sync_copy(batchf_hbm.at[wid], idsf_v)

        lanes = lax.iota(jnp.int32, L)

        def rle_step(i, prev):
            base = pl.multiple_of(i * L, L)
            a = idsf_v[pl.ds(base, L)]
            gpos = base + lanes
            nxt = plsc.load_gather(idsf_v, [jnp.minimum(gpos + 1, RPW - 1)])
            valid = gpos < RPW
            is_end = ((a != nxt) | (gpos == RPW - 1)) & valid
            e = jnp.where(is_end, gpos, -1)
            cm = plsc.cummax(e)
            cm_v[...] = cm
            shifted = plsc.load_gather(cm_v, [jnp.maximum(lanes - 1, 0)])
            prev_v = jnp.where(lanes == 0, prev, shifted)
            prev_v = jnp.maximum(prev_v, prev)
            runlen = (gpos - prev_v).astype(jnp.float32)
            plsc.addupdate_scatter(hist_v, [a], runlen, mask=is_end)
            return jnp.maximum(prev, jnp.max(e))

        lax.fori_loop(0, NVEC, rle_step, jnp.int32(-1))
        pltpu.sync_copy(hist_v, cnt_hbm.at[wid])

        plsc.subcore_barrier()

        for j in range(NBLK):
            in_dma[j].wait()
            start_sc(j)
            sc_dma[j].wait()
            if j + 2 < NBLK:
                start_in(j + 2)

        plsc.subcore_barrier()

        pltpu.sync_copy(acc_sh.at[pl.ds(s * STRIPE, STRIPE)], zed_v)
        pltpu.sync_copy(zed_v, sum_hbm.at[c, pl.ds(s * STRIPE, STRIPE)])

    return k(node_r, batch_r, batch_f, zeros_d)


def _onehot_partial_tc(node_tc, ids_tc):
    def body(ids_ref, node_ref, out_ref, acc_ref):
        i = pl.program_id(0)

        @pl.when(i == 0)
        def _():
            acc_ref[...] = jnp.zeros_like(acc_ref)

        ids = ids_ref[0]
        gi = lax.broadcasted_iota(jnp.int32, (G, TCB), 0)
        oh = jnp.where(ids == gi, 1.0, 0.0)
        acc_ref[...] += lax.dot_general(
            oh, node_ref[...], (((1,), (0,)), ((), ())),
            preferred_element_type=jnp.float32)

        @pl.when(i == NTCB - 1)
        def _():
            out_ref[...] = acc_ref[...]

    return pl.pallas_call(
        body,
        grid=(NTCB,),
        in_specs=[
            pl.BlockSpec((1, 1, TCB), lambda i: (i, 0, 0)),
            pl.BlockSpec((TCB, D), lambda i: (i, 0)),
        ],
        out_specs=pl.BlockSpec((G, D), lambda i: (0, 0)),
        out_shape=jax.ShapeDtypeStruct((G, D), jnp.float32),
        scratch_shapes=[pltpu.VMEM((G, D), jnp.float32)],
    )(ids_tc, node_tc)


def _mlp_tc(parts, tc_part, cnt, W1, b1, W2, b2, Wout, bout):
    def body(part_ref, tcp_ref, cnt_ref, ones_ref, w1_ref, b1_ref, w2_ref,
             b2_ref, wo_ref, bo_ref, out_ref):
        seg = part_ref[0] + part_ref[1] + tcp_ref[...]
        cnt_col = lax.dot_general(
            cnt_ref[...], ones_ref[...], (((0,), (0,)), ((), ())),
            preferred_element_type=jnp.float32,
            precision=lax.Precision.HIGHEST)
        h = seg / jnp.maximum(cnt_col, 1.0)
        h = jax.nn.softplus(
            lax.dot_general(h, w1_ref[...], (((1,), (1,)), ((), ())),
                            preferred_element_type=jnp.float32) + b1_ref[...])
        h = jax.nn.softplus(
            lax.dot_general(h, w2_ref[...], (((1,), (1,)), ((), ())),
                            preferred_element_type=jnp.float32) + b2_ref[...])
        hh = h.astype(jnp.bfloat16).astype(jnp.float32)
        ww = wo_ref[...].astype(jnp.bfloat16).astype(jnp.float32)
        out = jnp.sum(hh * ww, axis=1, keepdims=True) + bo_ref[0, 0]
        out_ref[...] = out

    ones_col = jnp.ones((NW, 1), jnp.float32)
    return pl.pallas_call(
        body,
        out_shape=jax.ShapeDtypeStruct((G, 1), jnp.float32),
    )(parts, tc_part, cnt, ones_col, W1, b1.reshape(1, D),
      W2, b2.reshape(1, D), Wout, bout.reshape(1, 1))


def kernel(node_attr, batch, W1, b1, W2, b2, Wout, bout):
    batch_i = batch.astype(jnp.int32)
    node_r = node_attr[:N_SC].reshape(NW, NBLK, BLK, D)
    batch_r = batch_i[:N_SC].reshape(NW, NBLK, BLK)
    batch_f = jnp.pad(batch_i.reshape(NW, RPW), ((0, 0), (0, IDS_PAD - RPW)))
    zeros_d = jnp.zeros((STRIPE, D), jnp.float32)
    node_tc = node_attr[N_SC:]
    ids_tc = batch_i[N_SC:].reshape(NTCB, 1, TCB)
    parts, cnt = _seg_pool_sc(node_r, batch_r, batch_f, zeros_d)
    tc_part = _onehot_partial_tc(node_tc, ids_tc)
    out = _mlp_tc(parts, tc_part, cnt, W1, b1, W2, b2, Wout, bout)
    return out.reshape(-1)


# speedup vs baseline: 1.0398x; 1.0396x over previous
import dataclasses
import functools

import jax
import jax.numpy as jnp
from jax import lax
from jax.experimental import pallas as pl
from jax.experimental.pallas import tpu as pltpu
from jax.experimental.pallas import tpu_sc as plsc

N = 100000
D = 128
G = 512
NC = 2
NS = 16
NW = NC * NS
RPW = N // NW
STRIPE = G // NS
L = 16
NVEC = (RPW + L - 1) // L
IDS_PAD = NVEC * L

BLK = 125
NBLK = RPW // BLK
RLE_CHUNK = (NVEC + NBLK - 1) // NBLK


def _seg_pool_sc(node_r, batch_r, batch_f, zeros_d):
    mesh = plsc.VectorSubcoreMesh(core_axis_name="c", subcore_axis_name="s")
    cp = pltpu.CompilerParams()
    if "needs_layout_passes" in pltpu.CompilerParams.__dataclass_fields__:
        cp = dataclasses.replace(cp, needs_layout_passes=False)

    @functools.partial(
        pl.kernel,
        compiler_params=cp,
        out_type=(
            jax.ShapeDtypeStruct((NC, G, D), jnp.float32),
            jax.ShapeDtypeStruct((NW, G), jnp.float32),
        ),
        mesh=mesh,
        scratch_types=[
            pltpu.VMEM((NBLK, BLK), jnp.int32),
            pltpu.VMEM((IDS_PAD,), jnp.int32),
            pltpu.VMEM((BLK, D), jnp.float32),
            pltpu.VMEM((BLK, D), jnp.float32),
            pltpu.VMEM((STRIPE, D), jnp.float32),
            pltpu.VMEM((G,), jnp.float32),
            pltpu.VMEM((L,), jnp.int32),
            pltpu.VMEM_SHARED((G, D), jnp.float32),
            pltpu.SemaphoreType.DMA,
            pltpu.SemaphoreType.DMA,
            pltpu.SemaphoreType.DMA,
            pltpu.SemaphoreType.DMA,
        ],
    )
    def k(node_hbm, batch_hbm, batchf_hbm, zd_hbm, sum_hbm, cnt_hbm,
          ids_v, idsf_v, buf0_v, buf1_v, zed_v, hist_v, cm_v, acc_sh,
          sin0, sin1, ssc0, ssc1):
        c = lax.axis_index("c")
        s = lax.axis_index("s")
        wid = c * NS + s
        bufs = (buf0_v, buf1_v)
        sins = (sin0, sin1)
        sscs = (ssc0, ssc1)

        in_dma = {}
        sc_dma = {}

        def start_in(j):
            in_dma[j] = pltpu.async_copy(node_hbm.at[wid, j], bufs[j % 2],
                                         sins[j % 2])

        def start_sc(j):
            sc_dma[j] = pltpu.async_copy(bufs[j % 2], acc_sh.at[ids_v.at[j]],
                                         sscs[j % 2], add=True)

        start_in(0)
        start_in(1)

        pltpu.sync_copy(zd_hbm, zed_v)
        pltpu.sync_copy(zed_v, acc_sh.at[pl.ds(s * STRIPE, STRIPE)])
        zf = jnp.zeros((L,), jnp.float32)

        @pl.loop(0, G // L)
        def _(i):
            hist_v[pl.ds(i * L, L)] = zf

        pltpu.sync_copy(batch_hbm.at[wid], ids_v)
        pltpu.sync_copy(batchf_hbm.at[wid], idsf_v)

        lanes = lax.iota(jnp.int32, L)

        def rle_step(i, prev):
            base = pl.multiple_of(i * L, L)
            a = idsf_v[pl.ds(base, L)]
            gpos = base + lanes
            nxt = plsc.load_gather(idsf_v, [jnp.minimum(gpos + 1, RPW - 1)])
            valid = gpos < RPW
            is_end = ((a != nxt) | (gpos == RPW - 1)) & valid
            e = jnp.where(is_end, gpos, -1)
            cm = plsc.cummax(e)
            cm_v[...] = cm
            shifted = plsc.load_gather(cm_v, [jnp.maximum(lanes - 1, 0)])
            prev_v = jnp.where(lanes == 0, prev, shifted)
            prev_v = jnp.maximum(prev_v, prev)
            runlen = (gpos - prev_v).astype(jnp.float32)
            plsc.addupdate_scatter(hist_v, [a], runlen, mask=is_end)
            return jnp.maximum(prev, jnp.max(e))

        plsc.subcore_barrier()

        prev = jnp.int32(-1)
        for j in range(NBLK):
            in_dma[j].wait()
            start_sc(j)
            lo = j * RLE_CHUNK
            hi = min((j + 1) * RLE_CHUNK, NVEC)
            if lo < NVEC:
                prev = lax.fori_loop(lo, hi, rle_step, prev)
            sc_dma[j].wait()
            if j + 2 < NBLK:
                start_in(j + 2)
        pltpu.sync_copy(hist_v, cnt_hbm.at[wid])

        plsc.subcore_barrier()

        pltpu.sync_copy(acc_sh.at[pl.ds(s * STRIPE, STRIPE)], zed_v)
        pltpu.sync_copy(zed_v, sum_hbm.at[c, pl.ds(s * STRIPE, STRIPE)])

    return k(node_r, batch_r, batch_f, zeros_d)


def _mlp_tc(parts, cnt, W1, b1, W2, b2, Wout, bout):
    def body(part_ref, cnt_ref, ones_ref, w1_ref, b1_ref, w2_ref,
             b2_ref, wo_ref, bo_ref, out_ref):
        seg = part_ref[0] + part_ref[1]
        cnt_col = lax.dot_general(
            cnt_ref[...], ones_ref[...], (((0,), (0,)), ((), ())),
            preferred_element_type=jnp.float32,
            precision=lax.Precision.HIGHEST)
        h = seg / jnp.maximum(cnt_col, 1.0)
        h = jax.nn.softplus(
            lax.dot_general(h, w1_ref[...], (((1,), (1,)), ((), ())),
                            preferred_element_type=jnp.float32) + b1_ref[...])
        h = jax.nn.softplus(
            lax.dot_general(h, w2_ref[...], (((1,), (1,)), ((), ())),
                            preferred_element_type=jnp.float32) + b2_ref[...])
        hh = h.astype(jnp.bfloat16).astype(jnp.float32)
        ww = wo_ref[...].astype(jnp.bfloat16).astype(jnp.float32)
        out = jnp.sum(hh * ww, axis=1, keepdims=True) + bo_ref[0, 0]
        out_ref[...] = out

    ones_col = jnp.ones((NW, 1), jnp.float32)
    return pl.pallas_call(
        body,
        out_shape=jax.ShapeDtypeStruct((G, 1), jnp.float32),
    )(parts, cnt, ones_col, W1, b1.reshape(1, D),
      W2, b2.reshape(1, D), Wout, bout.reshape(1, 1))


def kernel(node_attr, batch, W1, b1, W2, b2, Wout, bout):
    batch_i = batch.astype(jnp.int32)
    node_r = node_attr.reshape(NW, NBLK, BLK, D)
    batch_r = batch_i.reshape(NW, NBLK, BLK)
    batch_f = jnp.pad(batch_i.reshape(NW, RPW), ((0, 0), (0, IDS_PAD - RPW)))
    zeros_d = jnp.zeros((STRIPE, D), jnp.float32)
    parts, cnt = _seg_pool_sc(node_r, batch_r, batch_f, zeros_d)
    out = _mlp_tc(parts, cnt, W1, b1, W2, b2, Wout, bout)
    return out.reshape(-1)


# speedup vs baseline: 1.0614x; 1.0208x over previous
import dataclasses
import functools

import jax
import jax.numpy as jnp
from jax import lax
from jax.experimental import pallas as pl
from jax.experimental.pallas import tpu as pltpu
from jax.experimental.pallas import tpu_sc as plsc

N = 100000
D = 128
G = 512
NC = 2
NS = 16
NW = NC * NS
RPW = N // NW
STRIPE = G // NS
L = 16
NVEC = (RPW + L - 1) // L
IDS_PAD = NVEC * L

BLK = 125
NBLK = RPW // BLK
RLE_CHUNK = (NVEC + NBLK - 1) // NBLK


def _seg_pool_sc(node_r, batch_r, batch_f, zeros_d):
    mesh = plsc.VectorSubcoreMesh(core_axis_name="c", subcore_axis_name="s")
    cp = pltpu.CompilerParams()
    if "needs_layout_passes" in pltpu.CompilerParams.__dataclass_fields__:
        cp = dataclasses.replace(cp, needs_layout_passes=False)

    @functools.partial(
        pl.kernel,
        compiler_params=cp,
        out_type=(
            jax.ShapeDtypeStruct((NC, G, D), jnp.float32),
            jax.ShapeDtypeStruct((NW, G), jnp.float32),
        ),
        mesh=mesh,
        scratch_types=[
            pltpu.VMEM((NBLK, BLK), jnp.int32),
            pltpu.VMEM((IDS_PAD,), jnp.int32),
            pltpu.VMEM((BLK, D), jnp.float32),
            pltpu.VMEM((BLK, D), jnp.float32),
            pltpu.VMEM((BLK, D), jnp.float32),
            pltpu.VMEM((BLK, D), jnp.float32),
            pltpu.VMEM((STRIPE, D), jnp.float32),
            pltpu.VMEM((G,), jnp.float32),
            pltpu.VMEM((L,), jnp.int32),
            pltpu.VMEM_SHARED((G, D), jnp.float32),
            pltpu.SemaphoreType.DMA,
            pltpu.SemaphoreType.DMA,
            pltpu.SemaphoreType.DMA,
            pltpu.SemaphoreType.DMA,
            pltpu.SemaphoreType.DMA,
            pltpu.SemaphoreType.DMA,
            pltpu.SemaphoreType.DMA,
            pltpu.SemaphoreType.DMA,
        ],
    )
    def k(node_hbm, batch_hbm, batchf_hbm, zd_hbm, sum_hbm, cnt_hbm,
          ids_v, idsf_v, buf0_v, buf1_v, buf2_v, buf3_v, zed_v, hist_v, cm_v,
          acc_sh, sin0, sin1, sin2, sin3, ssc0, ssc1, ssc2, ssc3):
        c = lax.axis_index("c")
        s = lax.axis_index("s")
        wid = c * NS + s
        bufs = (buf0_v, buf1_v, buf2_v, buf3_v)
        sins = (sin0, sin1, sin2, sin3)
        sscs = (ssc0, ssc1, ssc2, ssc3)

        in_dma = {}
        sc_dma = {}

        def start_in(j):
            in_dma[j] = pltpu.async_copy(node_hbm.at[wid, j], bufs[j % 4],
                                         sins[j % 4])

        def start_sc(j):
            sc_dma[j] = pltpu.async_copy(bufs[j % 4], acc_sh.at[ids_v.at[j]],
                                         sscs[j % 4], add=True)

        start_in(0)
        start_in(1)
        start_in(2)

        pltpu.sync_copy(zd_hbm, zed_v)
        pltpu.sync_copy(zed_v, acc_sh.at[pl.ds(s * STRIPE, STRIPE)])
        zf = jnp.zeros((L,), jnp.float32)

        @pl.loop(0, G // L)
        def _(i):
            hist_v[pl.ds(i * L, L)] = zf

        pltpu.sync_copy(batch_hbm.at[wid], ids_v)
        pltpu.sync_copy(batchf_hbm.at[wid], idsf_v)

        lanes = lax.iota(jnp.int32, L)

        def rle_step(i, prev):
            base = pl.multiple_of(i * L, L)
            a = idsf_v[pl.ds(base, L)]
            gpos = base + lanes
            nxt = plsc.load_gather(idsf_v, [jnp.minimum(gpos + 1, RPW - 1)])
            valid = gpos < RPW
            is_end = ((a != nxt) | (gpos == RPW - 1)) & valid
            e = jnp.where(is_end, gpos, -1)
            cm = plsc.cummax(e)
            cm_v[...] = cm
            shifted = plsc.load_gather(cm_v, [jnp.maximum(lanes - 1, 0)])
            prev_v = jnp.where(lanes == 0, prev, shifted)
            prev_v = jnp.maximum(prev_v, prev)
            runlen = (gpos - prev_v).astype(jnp.float32)
            plsc.addupdate_scatter(hist_v, [a], runlen, mask=is_end)
            return jnp.maximum(prev, jnp.max(e))

        plsc.subcore_barrier()

        prev = jnp.int32(-1)
        for j in range(NBLK):
            in_dma[j].wait()
            start_sc(j)
            lo = j * RLE_CHUNK
            hi = min((j + 1) * RLE_CHUNK, NVEC)
            if lo < NVEC:
                prev = lax.fori_loop(lo, hi, rle_step, prev)
            if j >= 1:
                sc_dma[j - 1].wait()
            if j + 3 < NBLK:
                start_in(j + 3)
        sc_dma[NBLK - 1].wait()
        pltpu.sync_copy(hist_v, cnt_hbm.at[wid])

        plsc.subcore_barrier()

        pltpu.sync_copy(acc_sh.at[pl.ds(s * STRIPE, STRIPE)], zed_v)
        pltpu.sync_copy(zed_v, sum_hbm.at[c, pl.ds(s * STRIPE, STRIPE)])

    return k(node_r, batch_r, batch_f, zeros_d)


def _mlp_tc(parts, cnt, W1, b1, W2, b2, Wout, bout):
    def body(part_ref, cnt_ref, ones_ref, w1_ref, b1_ref, w2_ref,
             b2_ref, wo_ref, bo_ref, out_ref):
        seg = part_ref[0] + part_ref[1]
        cnt_col = lax.dot_general(
            cnt_ref[...], ones_ref[...], (((0,), (0,)), ((), ())),
            preferred_element_type=jnp.float32,
            precision=lax.Precision.HIGHEST)
        h = seg / jnp.maximum(cnt_col, 1.0)
        h = jax.nn.softplus(
            lax.dot_general(h, w1_ref[...], (((1,), (1,)), ((), ())),
                            preferred_element_type=jnp.float32) + b1_ref[...])
        h = jax.nn.softplus(
            lax.dot_general(h, w2_ref[...], (((1,), (1,)), ((), ())),
                            preferred_element_type=jnp.float32) + b2_ref[...])
        hh = h.astype(jnp.bfloat16).astype(jnp.float32)
        ww = wo_ref[...].astype(jnp.bfloat16).astype(jnp.float32)
        out = jnp.sum(hh * ww, axis=1, keepdims=True) + bo_ref[0, 0]
        out_ref[...] = out

    ones_col = jnp.ones((NW, 1), jnp.float32)
    return pl.pallas_call(
        body,
        out_shape=jax.ShapeDtypeStruct((G, 1), jnp.float32),
    )(parts, cnt, ones_col, W1, b1.reshape(1, D),
      W2, b2.reshape(1, D), Wout, bout.reshape(1, 1))


def kernel(node_attr, batch, W1, b1, W2, b2, Wout, bout):
    batch_i = batch.astype(jnp.int32)
    node_r = node_attr.reshape(NW, NBLK, BLK, D)
    batch_r = batch_i.reshape(NW, NBLK, BLK)
    batch_f = jnp.pad(batch_i.reshape(NW, RPW), ((0, 0), (0, IDS_PAD - RPW)))
    zeros_d = jnp.zeros((STRIPE, D), jnp.float32)
    parts, cnt = _seg_pool_sc(node_r, batch_r, batch_f, zeros_d)
    out = _mlp_tc(parts, cnt, W1, b1, W2, b2, Wout, bout)
    return out.reshape(-1)
